# Initial kernel scaffold; baseline (speedup 1.0000x reference)
import jax
import jax.numpy as jnp
from jax.experimental import pallas as pl


def kernel(hidden_states, weight, e_score_correction_bias):
    raise NotImplementedError("write your pallas kernel here")


# speedup vs baseline: 3.3572x; 3.3572x over previous
import functools

import jax
import jax.numpy as jnp
from jax.experimental import pallas as pl
from jax.experimental.pallas import tpu as pltpu

NUM_EXPERTS = 64
TOP_K = 8
N_GROUP = 8
TOPK_GROUP = 4
EPG = NUM_EXPERTS // N_GROUP
ROUTED_SCALING = 2.5

TILE_T = 512


def _router_body(h_ref, w_ref, b_ref, rw_ref, idx_ref):
    t = h_ref.shape[0]
    logits = jax.lax.dot_general(
        w_ref[...], h_ref[...],
        dimension_numbers=(((1,), (1,)), ((), ())),
        preferred_element_type=jnp.float32,
    )
    s = jax.nn.sigmoid(logits)
    sfc = s + b_ref[...]

    g3 = sfc.reshape(N_GROUP, EPG, t)
    ii = jax.lax.broadcasted_iota(jnp.int32, (N_GROUP, EPG, t), 1)
    m1 = jnp.max(g3, axis=1)
    eq1 = g3 == m1[:, None, :]
    i1 = jnp.min(jnp.where(eq1, ii, EPG), axis=1)
    g3m = jnp.where(ii == i1[:, None, :], -jnp.inf, g3)
    m2 = jnp.max(g3m, axis=1)
    gs = m1 + m2

    ga = jax.lax.broadcasted_iota(jnp.int32, (N_GROUP, N_GROUP, 1), 0)
    gb = jax.lax.broadcasted_iota(jnp.int32, (N_GROUP, N_GROUP, 1), 1)
    sa = gs[:, None, :]
    sb = gs[None, :, :]
    beats = (sb > sa) | ((sb == sa) & (gb < ga))
    rank = jnp.sum(beats.astype(jnp.int32), axis=1)
    gmask = (rank < TOPK_GROUP).astype(jnp.float32)

    emask = jnp.broadcast_to(gmask[:, None, :], (N_GROUP, EPG, t))
    masked = sfc * emask.reshape(NUM_EXPERTS, t)

    ei = jax.lax.broadcasted_iota(jnp.int32, (NUM_EXPERTS, t), 0)
    idx_rows = []
    rw_rows = []
    for _ in range(TOP_K):
        m = jnp.max(masked, axis=0)
        eq = masked == m[None, :]
        sel = jnp.min(jnp.where(eq, ei, NUM_EXPERTS), axis=0)
        hit = ei == sel[None, :]
        rw_rows.append(jnp.sum(jnp.where(hit, s, 0.0), axis=0))
        masked = jnp.where(hit, -jnp.inf, masked)
        idx_rows.append(sel)

    rws = jnp.stack(rw_rows, axis=0)
    denom = jnp.sum(rws, axis=0) + 1e-20
    rw_ref[...] = rws * (ROUTED_SCALING / denom)[None, :]
    idx_ref[...] = jnp.stack(idx_rows, axis=0)


@functools.partial(jax.jit, static_argnames=())
def kernel(hidden_states, weight, e_score_correction_bias):
    T, H = hidden_states.shape
    E = weight.shape[0]
    n_tiles = T // TILE_T
    bias_col = e_score_correction_bias.reshape(E, 1)

    rw_t, idx_t = pl.pallas_call(
        _router_body,
        grid=(n_tiles,),
        in_specs=[
            pl.BlockSpec((TILE_T, H), lambda i: (i, 0)),
            pl.BlockSpec((E, H), lambda i: (0, 0)),
            pl.BlockSpec((E, 1), lambda i: (0, 0)),
        ],
        out_specs=[
            pl.BlockSpec((TOP_K, TILE_T), lambda i: (0, i)),
            pl.BlockSpec((TOP_K, TILE_T), lambda i: (0, i)),
        ],
        out_shape=[
            jax.ShapeDtypeStruct((TOP_K, T), jnp.float32),
            jax.ShapeDtypeStruct((TOP_K, T), jnp.int32),
        ],
    )(hidden_states, weight, bias_col)

    return rw_t.T, idx_t.T


# speedup vs baseline: 3.8615x; 1.1502x over previous
import functools

import jax
import jax.numpy as jnp
from jax.experimental import pallas as pl
from jax.experimental.pallas import tpu as pltpu

NUM_EXPERTS = 64
TOP_K = 8
N_GROUP = 8
TOPK_GROUP = 4
EPG = NUM_EXPERTS // N_GROUP
ROUTED_SCALING = 2.5

TILE_T = 512


def _router_body(h_ref, w_ref, b_ref, rw_ref, idx_ref):
    t = h_ref.shape[0]
    logits = jax.lax.dot_general(
        w_ref[...], h_ref[...],
        dimension_numbers=(((1,), (1,)), ((), ())),
        preferred_element_type=jnp.float32,
    )
    s = jax.nn.sigmoid(logits)
    sfc = s + b_ref[...]

    NEG = jnp.int32(-2147483647 - 1)
    ei = jax.lax.broadcasted_iota(jnp.int32, (NUM_EXPERTS, t), 0)
    qsfc = (sfc * 16777216.0).astype(jnp.int32)
    qkey = (qsfc << 6) | (63 - ei)

    k3 = qkey.reshape(N_GROUP, EPG, t)
    m1k = jnp.max(k3, axis=1)
    k3b = jnp.where(k3 == m1k[:, None, :], NEG, k3)
    m2k = jnp.max(k3b, axis=1)
    gi = jax.lax.broadcasted_iota(jnp.int32, (N_GROUP, t), 0)
    gkey = (((m1k >> 6) + (m2k >> 6)) << 3) | (7 - gi)

    gmask = jnp.zeros((N_GROUP, t), dtype=jnp.bool_)
    for _ in range(TOPK_GROUP):
        gm = jnp.max(gkey, axis=0)
        ghit = gkey == gm[None, :]
        gmask = gmask | ghit
        gkey = jnp.where(ghit, NEG, gkey)

    emask = jnp.broadcast_to(gmask[:, None, :], (N_GROUP, EPG, t))
    key = jnp.where(emask.reshape(NUM_EXPERTS, t), qkey, 63 - ei)

    idx_rows = []
    rw_rows = []
    for _ in range(TOP_K):
        kmax = jnp.max(key, axis=0)
        sel = 63 - (kmax & 63)
        hit = key == kmax[None, :]
        rw_rows.append(jnp.sum(jnp.where(hit, s, 0.0), axis=0))
        key = jnp.where(hit, NEG, key)
        idx_rows.append(sel)

    rws = jnp.stack(rw_rows, axis=0)
    denom = jnp.sum(rws, axis=0) + 1e-20
    rw_ref[...] = rws * (ROUTED_SCALING / denom)[None, :]
    idx_ref[...] = jnp.stack(idx_rows, axis=0)


@functools.partial(jax.jit, static_argnames=())
def kernel(hidden_states, weight, e_score_correction_bias):
    T, H = hidden_states.shape
    E = weight.shape[0]
    n_tiles = T // TILE_T
    bias_col = e_score_correction_bias.reshape(E, 1)

    rw_t, idx_t = pl.pallas_call(
        _router_body,
        grid=(n_tiles,),
        in_specs=[
            pl.BlockSpec((TILE_T, H), lambda i: (i, 0)),
            pl.BlockSpec((E, H), lambda i: (0, 0)),
            pl.BlockSpec((E, 1), lambda i: (0, 0)),
        ],
        out_specs=[
            pl.BlockSpec((TOP_K, TILE_T), lambda i: (0, i)),
            pl.BlockSpec((TOP_K, TILE_T), lambda i: (0, i)),
        ],
        out_shape=[
            jax.ShapeDtypeStruct((TOP_K, T), jnp.float32),
            jax.ShapeDtypeStruct((TOP_K, T), jnp.int32),
        ],
    )(hidden_states, weight, bias_col)

    return rw_t.T, idx_t.T


# speedup vs baseline: 4.9342x; 1.2778x over previous
import functools

import jax
import jax.numpy as jnp
from jax.experimental import pallas as pl
from jax.experimental.pallas import tpu as pltpu

NUM_EXPERTS = 64
TOP_K = 8
N_GROUP = 8
TOPK_GROUP = 4
EPG = NUM_EXPERTS // N_GROUP
ROUTED_SCALING = 2.5

TILE_T = 1024


def _router_body(h_ref, w_ref, b_ref, rw_ref, idx_ref):
    t = h_ref.shape[0]
    logits = jax.lax.dot_general(
        w_ref[...], h_ref[...],
        dimension_numbers=(((1,), (1,)), ((), ())),
        preferred_element_type=jnp.float32,
    )
    s = jax.nn.sigmoid(logits)
    sfc = s + b_ref[...]

    NEG = jnp.int32(-2147483647 - 1)
    ei = jax.lax.broadcasted_iota(jnp.int32, (NUM_EXPERTS, t), 0)
    qsfc = (sfc * 16777216.0).astype(jnp.int32)
    qkey = (qsfc << 6) | (63 - ei)

    k3 = qkey.reshape(N_GROUP, EPG, t)
    m1k = jnp.max(k3, axis=1)
    k3b = jnp.where(k3 == m1k[:, None, :], NEG, k3)
    m2k = jnp.max(k3b, axis=1)
    gi = jax.lax.broadcasted_iota(jnp.int32, (N_GROUP, t), 0)
    gkey = (((m1k >> 6) + (m2k >> 6)) << 3) | (7 - gi)

    gmask = jnp.zeros((N_GROUP, t), dtype=jnp.bool_)
    for _ in range(TOPK_GROUP):
        gm = jnp.max(gkey, axis=0)
        ghit = gkey == gm[None, :]
        gmask = gmask | ghit
        gkey = jnp.where(ghit, NEG, gkey)

    emask = jnp.broadcast_to(gmask[:, None, :], (N_GROUP, EPG, t))
    key = jnp.where(emask.reshape(NUM_EXPERTS, t), qkey, 63 - ei)

    idx_rows = []
    rw_rows = []
    for _ in range(TOP_K):
        kmax = jnp.max(key, axis=0)
        sel = 63 - (kmax & 63)
        hit = key == kmax[None, :]
        rw_rows.append(jnp.sum(jnp.where(hit, s, 0.0), axis=0))
        key = jnp.where(hit, NEG, key)
        idx_rows.append(sel)

    rws = jnp.stack(rw_rows, axis=0)
    denom = jnp.sum(rws, axis=0) + 1e-20
    rw_ref[...] = rws * (ROUTED_SCALING / denom)[None, :]
    idx_ref[...] = jnp.stack(idx_rows, axis=0)


@functools.partial(jax.jit, static_argnames=())
def kernel(hidden_states, weight, e_score_correction_bias):
    T, H = hidden_states.shape
    E = weight.shape[0]
    n_tiles = T // TILE_T
    bias_col = e_score_correction_bias.reshape(E, 1)

    rw_t, idx_t = pl.pallas_call(
        _router_body,
        grid=(n_tiles,),
        in_specs=[
            pl.BlockSpec((TILE_T, H), lambda i: (i, 0)),
            pl.BlockSpec((E, H), lambda i: (0, 0)),
            pl.BlockSpec((E, 1), lambda i: (0, 0)),
        ],
        out_specs=[
            pl.BlockSpec((TOP_K, TILE_T), lambda i: (0, i)),
            pl.BlockSpec((TOP_K, TILE_T), lambda i: (0, i)),
        ],
        out_shape=[
            jax.ShapeDtypeStruct((TOP_K, T), jnp.float32),
            jax.ShapeDtypeStruct((TOP_K, T), jnp.int32),
        ],
    )(hidden_states, weight, bias_col)

    return rw_t.T, idx_t.T


# speedup vs baseline: 5.4364x; 1.1018x over previous
import functools

import jax
import jax.numpy as jnp
from jax.experimental import pallas as pl
from jax.experimental.pallas import tpu as pltpu

NUM_EXPERTS = 64
TOP_K = 8
N_GROUP = 8
TOPK_GROUP = 4
EPG = NUM_EXPERTS // N_GROUP
ROUTED_SCALING = 2.5

TILE_T = 2048


def _router_body(h_ref, w_ref, b_ref, rw_ref, idx_ref):
    t = h_ref.shape[0]
    logits = jax.lax.dot_general(
        w_ref[...], h_ref[...],
        dimension_numbers=(((1,), (1,)), ((), ())),
        preferred_element_type=jnp.float32,
    )
    s = jax.nn.sigmoid(logits)
    sfc = s + b_ref[...]

    NEG = jnp.int32(-2147483647 - 1)
    ei = jax.lax.broadcasted_iota(jnp.int32, (NUM_EXPERTS, t), 0)
    qsfc = (sfc * 16777216.0).astype(jnp.int32)
    qkey = (qsfc << 6) | (63 - ei)

    k3 = qkey.reshape(N_GROUP, EPG, t)
    m1k = jnp.max(k3, axis=1)
    k3b = jnp.where(k3 == m1k[:, None, :], NEG, k3)
    m2k = jnp.max(k3b, axis=1)
    gi = jax.lax.broadcasted_iota(jnp.int32, (N_GROUP, t), 0)
    gkey = (((m1k >> 6) + (m2k >> 6)) << 3) | (7 - gi)

    gmask = jnp.zeros((N_GROUP, t), dtype=jnp.bool_)
    for _ in range(TOPK_GROUP):
        gm = jnp.max(gkey, axis=0)
        ghit = gkey == gm[None, :]
        gmask = gmask | ghit
        gkey = jnp.where(ghit, NEG, gkey)

    emask = jnp.broadcast_to(gmask[:, None, :], (N_GROUP, EPG, t))
    key = jnp.where(emask.reshape(NUM_EXPERTS, t), qkey, 63 - ei)

    idx_rows = []
    rw_rows = []
    for _ in range(TOP_K):
        kmax = jnp.max(key, axis=0)
        sel = 63 - (kmax & 63)
        hit = key == kmax[None, :]
        rw_rows.append(jnp.sum(jnp.where(hit, s, 0.0), axis=0))
        key = jnp.where(hit, NEG, key)
        idx_rows.append(sel)

    rws = jnp.stack(rw_rows, axis=0)
    denom = jnp.sum(rws, axis=0) + 1e-20
    rw_ref[...] = rws * (ROUTED_SCALING / denom)[None, :]
    idx_ref[...] = jnp.stack(idx_rows, axis=0)


@functools.partial(jax.jit, static_argnames=())
def kernel(hidden_states, weight, e_score_correction_bias):
    T, H = hidden_states.shape
    E = weight.shape[0]
    n_tiles = T // TILE_T
    bias_col = e_score_correction_bias.reshape(E, 1)

    rw_t, idx_t = pl.pallas_call(
        _router_body,
        grid=(n_tiles,),
        in_specs=[
            pl.BlockSpec((TILE_T, H), lambda i: (i, 0)),
            pl.BlockSpec((E, H), lambda i: (0, 0)),
            pl.BlockSpec((E, 1), lambda i: (0, 0)),
        ],
        out_specs=[
            pl.BlockSpec((TOP_K, TILE_T), lambda i: (0, i)),
            pl.BlockSpec((TOP_K, TILE_T), lambda i: (0, i)),
        ],
        out_shape=[
            jax.ShapeDtypeStruct((TOP_K, T), jnp.float32),
            jax.ShapeDtypeStruct((TOP_K, T), jnp.int32),
        ],
    )(hidden_states, weight, bias_col)

    return rw_t.T, idx_t.T


# speedup vs baseline: 5.9126x; 1.0876x over previous
import functools

import jax
import jax.numpy as jnp
from jax.experimental import pallas as pl
from jax.experimental.pallas import tpu as pltpu

NUM_EXPERTS = 64
TOP_K = 8
N_GROUP = 8
TOPK_GROUP = 4
EPG = NUM_EXPERTS // N_GROUP
ROUTED_SCALING = 2.5

TILE_T = 4096


def _router_body(h_ref, w_ref, b_ref, rw_ref, idx_ref):
    t = h_ref.shape[0]
    logits = jax.lax.dot_general(
        w_ref[...], h_ref[...],
        dimension_numbers=(((1,), (1,)), ((), ())),
        preferred_element_type=jnp.float32,
    )
    s = jax.nn.sigmoid(logits)
    sfc = s + b_ref[...]

    NEG = jnp.int32(-2147483647 - 1)
    ei = jax.lax.broadcasted_iota(jnp.int32, (NUM_EXPERTS, t), 0)
    qsfc = (sfc * 16777216.0).astype(jnp.int32)
    qkey = (qsfc << 6) | (63 - ei)

    k3 = qkey.reshape(N_GROUP, EPG, t)
    m1k = jnp.max(k3, axis=1)
    k3b = jnp.where(k3 == m1k[:, None, :], NEG, k3)
    m2k = jnp.max(k3b, axis=1)
    gi = jax.lax.broadcasted_iota(jnp.int32, (N_GROUP, t), 0)
    gkey = (((m1k >> 6) + (m2k >> 6)) << 3) | (7 - gi)

    gmask = jnp.zeros((N_GROUP, t), dtype=jnp.bool_)
    for _ in range(TOPK_GROUP):
        gm = jnp.max(gkey, axis=0)
        ghit = gkey == gm[None, :]
        gmask = gmask | ghit
        gkey = jnp.where(ghit, NEG, gkey)

    emask = jnp.broadcast_to(gmask[:, None, :], (N_GROUP, EPG, t))
    key = jnp.where(emask.reshape(NUM_EXPERTS, t), qkey, 63 - ei)

    idx_rows = []
    rw_rows = []
    for _ in range(TOP_K):
        kmax = jnp.max(key, axis=0)
        sel = 63 - (kmax & 63)
        hit = key == kmax[None, :]
        rw_rows.append(jnp.sum(jnp.where(hit, s, 0.0), axis=0))
        key = jnp.where(hit, NEG, key)
        idx_rows.append(sel)

    rws = jnp.stack(rw_rows, axis=0)
    denom = jnp.sum(rws, axis=0) + 1e-20
    rw_ref[...] = rws * (ROUTED_SCALING / denom)[None, :]
    idx_ref[...] = jnp.stack(idx_rows, axis=0)


@functools.partial(jax.jit, static_argnames=())
def kernel(hidden_states, weight, e_score_correction_bias):
    T, H = hidden_states.shape
    E = weight.shape[0]
    n_tiles = T // TILE_T
    bias_col = e_score_correction_bias.reshape(E, 1)

    rw_t, idx_t = pl.pallas_call(
        _router_body,
        grid=(n_tiles,),
        in_specs=[
            pl.BlockSpec((TILE_T, H), lambda i: (i, 0)),
            pl.BlockSpec((E, H), lambda i: (0, 0)),
            pl.BlockSpec((E, 1), lambda i: (0, 0)),
        ],
        out_specs=[
            pl.BlockSpec((TOP_K, TILE_T), lambda i: (0, i)),
            pl.BlockSpec((TOP_K, TILE_T), lambda i: (0, i)),
        ],
        out_shape=[
            jax.ShapeDtypeStruct((TOP_K, T), jnp.float32),
            jax.ShapeDtypeStruct((TOP_K, T), jnp.int32),
        ],
    )(hidden_states, weight, bias_col)

    return rw_t.T, idx_t.T
